# baseline (speedup 1.0000x reference)
import functools

import jax
import jax.numpy as jnp
from jax import lax
from jax.experimental import pallas as pl
from jax.experimental.pallas import tpu as pltpu
from jax.experimental.pallas import tpu_sc as plsc

_DIM = 16
_G = 128
_CH = 8


@functools.lru_cache(maxsize=None)
def _build_gather(total: int, vocab: int):
    info = plsc.get_sparse_core_info()
    nc, ns = info.num_cores, info.num_subcores
    nw = nc * ns
    npw = total // nw
    assert npw * nw == total and npw % (_G * _CH) == 0
    ng = npw // _G
    nchunk = ng // _CH
    rows = _CH * _G

    mesh = plsc.VectorSubcoreMesh(core_axis_name="c", subcore_axis_name="s")

    @functools.partial(
        pl.kernel,
        mesh=mesh,
        compiler_params=pltpu.CompilerParams(use_tc_tiling_on_sc=False),
        out_type=[
            jax.ShapeDtypeStruct((nw, npw, _DIM), jnp.float32),
            jax.ShapeDtypeStruct((nw, npw, _DIM), jnp.float32),
        ],
        scratch_types=[
            pltpu.VMEM((ng, _G), jnp.int32),
            pltpu.VMEM((rows, _DIM), jnp.float32),
            pltpu.VMEM((rows, _DIM), jnp.float32),
            pltpu.SemaphoreType.DMA,
            pltpu.SemaphoreType.DMA,
        ],
    )
    def gather_kernel(ids_hbm, embed_hbm, imag_hbm, out_r, out_i,
                      idx_v, real_v, imag_v, sem_r, sem_i):
        wid = lax.axis_index("s") * nc + lax.axis_index("c")
        pltpu.sync_copy(ids_hbm.at[wid], idx_v)

        def chunk_body(c, carry):
            waits = []
            for j in range(_CH):
                g = c * _CH + j
                dst = pl.ds(j * _G, _G)
                waits.append(pltpu.async_copy(
                    embed_hbm.at[idx_v.at[g]], real_v.at[dst], sem_r))
                waits.append(pltpu.async_copy(
                    imag_hbm.at[idx_v.at[g]], imag_v.at[dst], sem_i))
            for w in waits:
                w.wait()
            base = pl.ds(c * rows, rows)
            pltpu.sync_copy(real_v, out_r.at[wid, base])
            pltpu.sync_copy(imag_v, out_i.at[wid, base])
            return carry

        lax.fori_loop(0, nchunk, chunk_body, 0)

    return gather_kernel, nw, npw, ng


def kernel(ids, embed, imag_embed):
    b, l = ids.shape
    total = b * l
    vocab = embed.shape[0]
    gather_kernel, nw, npw, ng = _build_gather(total, vocab)
    ids_w = ids.reshape(nw, ng, _G).astype(jnp.int32)
    out_r, out_i = gather_kernel(ids_w, embed, imag_embed)
    r_flat, i_flat = lax.optimization_barrier(
        (out_r.reshape(-1), out_i.reshape(-1)))
    flat = lax.optimization_barrier(lax.complex(r_flat, i_flat))
    return flat.reshape(b, l, _DIM)


# speedup vs baseline: 3.1330x; 3.1330x over previous
import functools

import jax
import jax.numpy as jnp
from jax import lax
from jax.experimental import pallas as pl
from jax.experimental.pallas import tpu as pltpu
from jax.experimental.pallas import tpu_sc as plsc

_DIM = 16
_G = 128
_CH = 8


@functools.lru_cache(maxsize=None)
def _build_gather(total: int, vocab: int):
    info = plsc.get_sparse_core_info()
    nc, ns = info.num_cores, info.num_subcores
    nw = nc * ns
    npw = total // nw
    assert npw * nw == total and npw % (_G * _CH) == 0
    ng = npw // _G
    nchunk = ng // _CH
    rows = _CH * _G

    mesh = plsc.VectorSubcoreMesh(core_axis_name="c", subcore_axis_name="s")

    @functools.partial(
        pl.kernel,
        mesh=mesh,
        compiler_params=pltpu.CompilerParams(use_tc_tiling_on_sc=False),
        out_type=[
            jax.ShapeDtypeStruct((nw, npw, _DIM), jnp.float32),
            jax.ShapeDtypeStruct((nw, npw, _DIM), jnp.float32),
        ],
        scratch_types=[
            pltpu.VMEM((ng, _G), jnp.int32),
            pltpu.VMEM((rows, _DIM), jnp.float32),
            pltpu.VMEM((rows, _DIM), jnp.float32),
            pltpu.SemaphoreType.DMA,
            pltpu.SemaphoreType.DMA,
        ],
    )
    def gather_kernel(ids_hbm, embed_hbm, imag_hbm, out_r, out_i,
                      idx_v, real_v, imag_v, sem_r, sem_i):
        wid = lax.axis_index("s") * nc + lax.axis_index("c")
        pltpu.sync_copy(ids_hbm.at[wid], idx_v)

        def chunk_body(c, carry):
            waits = []
            for j in range(_CH):
                g = c * _CH + j
                dst = pl.ds(j * _G, _G)
                waits.append(pltpu.async_copy(
                    embed_hbm.at[idx_v.at[g]], real_v.at[dst], sem_r))
                waits.append(pltpu.async_copy(
                    imag_hbm.at[idx_v.at[g]], imag_v.at[dst], sem_i))
            for w in waits:
                w.wait()
            base = pl.ds(c * rows, rows)
            pltpu.sync_copy(real_v, out_r.at[wid, base])
            pltpu.sync_copy(imag_v, out_i.at[wid, base])
            return carry

        lax.fori_loop(0, nchunk, chunk_body, 0)

    return gather_kernel, nw, npw, ng


def kernel(ids, embed, imag_embed):
    b, l = ids.shape
    total = b * l
    vocab = embed.shape[0]
    gather_kernel, nw, npw, ng = _build_gather(total, vocab)
    ids_u = ids.T.reshape(nw, ng, _G).astype(jnp.int32)
    out_r, out_i = gather_kernel(ids_u, embed, imag_embed)
    r_t = lax.transpose(out_r.reshape(l, b, _DIM), (0, 2, 1))
    i_t = lax.transpose(out_i.reshape(l, b, _DIM), (0, 2, 1))
    r_t, i_t = lax.optimization_barrier((r_t, i_t))
    c_t = lax.optimization_barrier(lax.complex(r_t, i_t))
    return lax.transpose(c_t, (2, 0, 1))
